# baseline (speedup 1.0000x reference)
import jax
import jax.numpy as jnp
from jax import lax
from jax.experimental import pallas as pl
from jax.experimental.pallas import tpu as pltpu
from jax.experimental.pallas import tpu_sc as plsc

B, T, F = 32, 8192, 256
L = 16
NCH = T // L
UNROLL = 8
NACC = 4

_NEG = float("-inf")
_POS = float("inf")


def _argred(v, i, tf, ti, lane):
  for s in (8, 4, 2, 1):
    tf[...] = v
    ti[...] = i
    ov = plsc.load_gather(tf, [lane ^ s])
    oi = plsc.load_gather(ti, [lane ^ s])
    m = (ov > v) | ((ov == v) & (oi < i))
    v = jnp.where(m, ov, v)
    i = jnp.where(m, oi, i)
  return v, i


def _butterfly(x, tmp_ref, combine, lane):
  for s in (8, 4, 2, 1):
    tmp_ref[...] = x
    x = combine(x, plsc.load_gather(tmp_ref, [lane ^ s]))
  return x


def _topk(key_ref, k, lane, g, base, tf, ti, produce=None, src_ref=None):
  def sweep(i, carry):
    bv, bi, ci = carry
    for u in range(UNROLL):
      sl = pl.ds((i * UNROLL + u) * L, L)
      if produce is not None:
        x = produce(src_ref[sl])
        key_ref[sl] = x
      else:
        x = key_ref[sl]
      m = x > bv
      bv = jnp.maximum(bv, x)
      bi = jnp.where(m, ci, bi)
      ci = ci + L
    return bv, bi, ci

  bv, bi, _ = lax.fori_loop(
      0, NCH // UNROLL, sweep,
      (jnp.full((L,), _NEG, jnp.float32), jnp.zeros((L,), jnp.int32), lane))

  def extract(j, carry):
    bv, bi, g = carry
    mx, gi = _argred(bv, bi, tf, ti, lane)
    g = jnp.where(lane == base + j, gi, g)
    plsc.store_scatter(key_ref, [gi], jnp.full((L,), _NEG, jnp.float32),
                       mask=lane == 0)
    lstar = gi & (L - 1)
    idx0 = lstar + L * lane
    accs = [(jnp.full((L,), _NEG, jnp.float32), jnp.zeros((L,), jnp.int32))
            for _ in range(NACC)]
    for t in range(NCH // L):
      a = t % NACC
      cv, cbi = accs[a]
      idx = idx0 + t * (L * L)
      x = plsc.load_gather(key_ref, [idx])
      m = x > cv
      accs[a] = (jnp.maximum(cv, x), jnp.where(m, idx, cbi))
    cv, cbi = accs[0]
    for a in range(1, NACC):
      ov, oi = accs[a]
      m = (ov > cv) | ((ov == cv) & (oi < cbi))
      cv = jnp.where(m, ov, cv)
      cbi = jnp.where(m, oi, cbi)
    mxc, gic = _argred(cv, cbi, tf, ti, lane)
    bv = jnp.where(lane == lstar, mxc, bv)
    bi = jnp.where(lane == lstar, gic, bi)
    return bv, bi, g

  _, _, g = lax.fori_loop(0, k, extract, (bv, bi, g))
  return g


def _body(feat_hbm, srgb_hbm, sflow_hbm,
          out_nor, out_abn, out_cnor, out_cabn,
          s_v, f_v, k_v, tf, ti, h_ref, idx_v, d_v, rows_v, sem):
  lane = lax.iota(jnp.int32, L)
  wid = lax.axis_index("c") * 16 + lax.axis_index("s")

  pltpu.sync_copy(srgb_hbm.at[wid], s_v)
  pltpu.sync_copy(sflow_hbm.at[wid], f_v)

  h = _topk(k_v, 10, lane, jnp.zeros((L,), jnp.int32), 0, tf, ti,
            produce=lambda x: -jnp.abs(x - 0.5), src_ref=s_v)
  h_ref[...] = h
  flow_h = plsc.load_gather(f_v, [h])

  g = jnp.zeros((L,), jnp.int32)

  v = jnp.where(lane < 10, flow_h, _NEG)
  for j in range(3):
    mx, p = _argred(v, lane, tf, ti, lane)
    t = plsc.load_gather(h_ref, [p])
    g = jnp.where(lane == 1 + j, t, g)
    v = jnp.where(lane == p, _NEG, v)

  v2 = jnp.where(lane < 10, -flow_h, _NEG)
  mn, p2 = _argred(v2, lane, tf, ti, lane)
  g = jnp.where(lane == 0, plsc.load_gather(h_ref, [p2]), g)

  g = _topk(k_v, 2, lane, g, 4, tf, ti, produce=lambda x: -x, src_ref=s_v)

  g = _topk(s_v, 10, lane, g, 6, tf, ti)

  h_ref[...] = g + wid * T
  for (o_ref, off, k) in ((out_nor, 0, 1), (out_abn, 1, 3),
                          (out_cnor, 4, 2), (out_cabn, 6, 10)):
    src_lane = jnp.minimum(lane + off, off + k - 1)
    idx_v[...] = plsc.load_gather(h_ref, [src_lane])
    d_v[...] = wid * k + jnp.minimum(lane, k - 1)
    pltpu.async_copy(feat_hbm.at[idx_v], rows_v, sem).wait()
    pltpu.async_copy(rows_v, o_ref.at[d_v], sem).wait()


@jax.jit
def kernel(feat, scores_rgb, scores_flow):
  feat_flat = feat.reshape(B * T, F)
  f32 = jnp.float32
  run = pl.kernel(
      _body,
      out_type=(
          jax.ShapeDtypeStruct((B * 1, F), f32),
          jax.ShapeDtypeStruct((B * 3, F), f32),
          jax.ShapeDtypeStruct((B * 2, F), f32),
          jax.ShapeDtypeStruct((B * 10, F), f32),
      ),
      mesh=plsc.VectorSubcoreMesh(core_axis_name="c", subcore_axis_name="s",
                                  num_cores=2, num_subcores=16),
      compiler_params=pltpu.CompilerParams(needs_layout_passes=False),
      scratch_types=[
          pltpu.VMEM((T,), f32),
          pltpu.VMEM((T,), f32),
          pltpu.VMEM((T,), f32),
          pltpu.VMEM((L,), f32),
          pltpu.VMEM((L,), jnp.int32),
          pltpu.VMEM((L,), jnp.int32),
          pltpu.VMEM((L,), jnp.int32),
          pltpu.VMEM((L,), jnp.int32),
          pltpu.VMEM((L, F), f32),
          pltpu.SemaphoreType.DMA,
      ],
  )
  o1, o2, o3, o4 = run(feat_flat, scores_rgb, scores_flow)
  return (o1.reshape(B, 1, F), o2.reshape(B, 3, F),
          o3.reshape(B, 2, F), o4.reshape(B, 10, F))


# speedup vs baseline: 1.2140x; 1.2140x over previous
import jax
import jax.numpy as jnp
from jax import lax
from jax.experimental import pallas as pl
from jax.experimental.pallas import tpu as pltpu
from jax.experimental.pallas import tpu_sc as plsc

B, T, F = 32, 8192, 256
L = 16
NCH = T // L
UNROLL = 8
NACC = 4

_NEG = float("-inf")
_POS = float("inf")


def _argred(v, i, tf, ti, lane):
  for s in (8, 4, 2, 1):
    tf[...] = v
    ti[...] = i
    ov = plsc.load_gather(tf, [lane ^ s])
    oi = plsc.load_gather(ti, [lane ^ s])
    m = (ov > v) | ((ov == v) & (oi < i))
    v = jnp.where(m, ov, v)
    i = jnp.where(m, oi, i)
  return v, i


def _butterfly(x, tmp_ref, combine, lane):
  for s in (8, 4, 2, 1):
    tmp_ref[...] = x
    x = combine(x, plsc.load_gather(tmp_ref, [lane ^ s]))
  return x


def _topk(key_ref, k, lane, g, base, tf, ti, produce=None, src_ref=None):
  def sweep(i, carry):
    bv, bi, ci = carry
    for u in range(UNROLL):
      sl = pl.ds((i * UNROLL + u) * L, L)
      if produce is not None:
        x = produce(src_ref[sl])
        key_ref[sl] = x
      else:
        x = key_ref[sl]
      m = x > bv
      bv = jnp.maximum(bv, x)
      bi = jnp.where(m, ci, bi)
      ci = ci + L
    return bv, bi, ci

  bv, bi, _ = lax.fori_loop(
      0, NCH // UNROLL, sweep,
      (jnp.full((L,), _NEG, jnp.float32), jnp.zeros((L,), jnp.int32), lane))

  def extract(j, carry):
    bv, bi, g = carry
    mx, gi = _argred(bv, bi, tf, ti, lane)
    g = jnp.where(lane == base + j, gi, g)
    plsc.store_scatter(key_ref, [gi], jnp.full((L,), _NEG, jnp.float32),
                       mask=lane == 0)
    lstar = gi & (L - 1)
    idx0 = lstar + L * lane
    accs = [(jnp.full((L,), _NEG, jnp.float32), jnp.zeros((L,), jnp.int32))
            for _ in range(NACC)]
    for t in range(NCH // L):
      a = t % NACC
      cv, cbi = accs[a]
      idx = idx0 + t * (L * L)
      x = plsc.load_gather(key_ref, [idx])
      m = x > cv
      accs[a] = (jnp.maximum(cv, x), jnp.where(m, idx, cbi))
    cv, cbi = accs[0]
    for a in range(1, NACC):
      ov, oi = accs[a]
      m = (ov > cv) | ((ov == cv) & (oi < cbi))
      cv = jnp.where(m, ov, cv)
      cbi = jnp.where(m, oi, cbi)
    mxc, gic = _argred(cv, cbi, tf, ti, lane)
    bv = jnp.where(lane == lstar, mxc, bv)
    bi = jnp.where(lane == lstar, gic, bi)
    return bv, bi, g

  _, _, g = lax.fori_loop(0, k, extract, (bv, bi, g))
  return g


def _body(feat_hbm, srgb_hbm, sflow_hbm, out_rows,
          s_v, f_v, k_v, tf, ti, h_ref, idx_v, rows_v, sem):
  lane = lax.iota(jnp.int32, L)
  wid = lax.axis_index("c") * 16 + lax.axis_index("s")

  pltpu.sync_copy(srgb_hbm.at[wid], s_v)
  pltpu.sync_copy(sflow_hbm.at[wid], f_v)

  h = _topk(k_v, 10, lane, jnp.zeros((L,), jnp.int32), 0, tf, ti,
            produce=lambda x: -jnp.abs(x - 0.5), src_ref=s_v)
  h_ref[...] = h
  flow_h = plsc.load_gather(f_v, [h])

  g = jnp.zeros((L,), jnp.int32)

  v = jnp.where(lane < 10, flow_h, _NEG)
  for j in range(3):
    mx, p = _argred(v, lane, tf, ti, lane)
    t = plsc.load_gather(h_ref, [p])
    g = jnp.where(lane == 1 + j, t, g)
    v = jnp.where(lane == p, _NEG, v)

  v2 = jnp.where(lane < 10, -flow_h, _NEG)
  mn, p2 = _argred(v2, lane, tf, ti, lane)
  g = jnp.where(lane == 0, plsc.load_gather(h_ref, [p2]), g)

  g = _topk(k_v, 2, lane, g, 4, tf, ti, produce=lambda x: -x, src_ref=s_v)

  g = _topk(s_v, 10, lane, g, 6, tf, ti)

  idx_v[...] = g + wid * T
  pltpu.async_copy(feat_hbm.at[idx_v], rows_v, sem).wait()
  pltpu.sync_copy(rows_v, out_rows.at[wid])


@jax.jit
def kernel(feat, scores_rgb, scores_flow):
  feat_flat = feat.reshape(B * T, F)
  f32 = jnp.float32
  run = pl.kernel(
      _body,
      out_type=jax.ShapeDtypeStruct((B, L, F), f32),
      mesh=plsc.VectorSubcoreMesh(core_axis_name="c", subcore_axis_name="s",
                                  num_cores=2, num_subcores=16),
      compiler_params=pltpu.CompilerParams(needs_layout_passes=False),
      scratch_types=[
          pltpu.VMEM((T,), f32),
          pltpu.VMEM((T,), f32),
          pltpu.VMEM((T,), f32),
          pltpu.VMEM((L,), f32),
          pltpu.VMEM((L,), jnp.int32),
          pltpu.VMEM((L,), jnp.int32),
          pltpu.VMEM((L,), jnp.int32),
          pltpu.VMEM((L, F), f32),
          pltpu.SemaphoreType.DMA,
      ],
  )
  rows = run(feat_flat, scores_rgb, scores_flow)
  return (rows[:, 0:1], rows[:, 1:4], rows[:, 4:6], rows[:, 6:16])
